# baseline (speedup 1.0000x reference)
import functools

import jax
import jax.numpy as jnp
from jax import lax
from jax.experimental import pallas as pl
from jax.experimental.pallas import tpu as pltpu
from jax.experimental.pallas import tpu_sc as plsc

N = 10000
E = 320000
D = 128
NC = 2
NS = 16
L = 16
C = 128
NCH = 160
SCH = 8
EPT = NCH * C
E_PAD = NS * EPT
N_T = 10112
RPT = N_T // NS
HR = 80
HRPT = 8
_ROW_CHUNKS = ((0, 128), (128, 128), (256, 128), (384, 128), (512, 120))


def _sc_body(y_hbm, pk_hbm, agg_hbm, deg_hbm,
             cbuf, gbuf, hist, rbuf, idx80, table, shacc, gsem, ssem):
  c = lax.axis_index("c")
  s = lax.axis_index("s")

  zeros16 = jnp.zeros((L,), jnp.float32)
  ones16 = jnp.ones((L,), jnp.float32)
  lanes = lax.iota(jnp.int32, L)

  def zhist(i, _):
    for g in range(D // L):
      hist[i, pl.ds(g * L, L)] = zeros16
    return 0

  lax.fori_loop(0, HR, zhist, 0)

  for i in range(HRPT):
    for g in range(D // L):
      rbuf[i, pl.ds(g * L, L)] = zeros16
  for k in range(HR // L):
    idx80[pl.ds(k * L, L)] = lanes + (k * L)

  hbase = s * HRPT

  @pl.when(s < HR // HRPT)
  def _zero_shacc():
    pltpu.sync_copy(rbuf, shacc.at[pl.ds(hbase, HRPT)])

  def zrow(i, _):
    for g in range(D // L):
      gbuf[0, i, pl.ds(g * L, L)] = zeros16
    return 0

  lax.fori_loop(0, C, zrow, 0)
  base = s * RPT
  for off, sz in _ROW_CHUNKS:
    pltpu.sync_copy(gbuf.at[0, pl.ds(0, sz)], table.at[pl.ds(base + off, sz)])

  plsc.subcore_barrier()

  def cidx_copy(j, q):
    pltpu.sync_copy(pk_hbm.at[c, s, j], cbuf.at[q, pl.ds(0, 3)])

  cidx_copy(0, 0)
  cidx_copy(1, 1)
  pltpu.async_copy(y_hbm.at[cbuf.at[0, 0]], gbuf.at[0], gsem.at[0])

  def chunk(j, _):
    p = lax.bitwise_and(j, 1)
    pn = 1 - p
    qj = lax.bitwise_and(j, 3)
    q1 = lax.bitwise_and(j + 1, 3)
    q2 = lax.bitwise_and(j + 2, 3)
    pltpu.make_async_copy(
        y_hbm.at[cbuf.at[qj, 0]], gbuf.at[p], gsem.at[p]).wait()

    @pl.when(j >= 1)
    def _drain_prev():
      pltpu.make_async_copy(
          gbuf.at[pn], table.at[cbuf.at[q1, 1]], ssem.at[pn]).wait()

    @pl.when(j + 1 < NCH)
    def _prefetch_gather():
      pltpu.async_copy(y_hbm.at[cbuf.at[q1, 0]], gbuf.at[pn], gsem.at[pn])

    def group(g, _):
      wv = plsc.bitcast(cbuf[qj, 2, pl.ds(g * L, L)], jnp.float32)
      iv = cbuf[qj, 1, pl.ds(g * L, L)]
      ir = lax.shift_right_logical(iv, 7)
      ic = lax.bitwise_and(iv, jnp.int32(D - 1))
      for k in range(L):
        i = g * L + k
        w = wv[k]
        for gg in range(D // L):
          gbuf[p, i, pl.ds(gg * L, L)] = gbuf[p, i, pl.ds(gg * L, L)] * w
      for k in range(L):
        plsc.addupdate_scatter(hist, [ir, ic], ones16, mask=lanes == k)
      return 0

    lax.fori_loop(0, C // L, group, 0)
    pltpu.async_copy(gbuf.at[p], table.at[cbuf.at[qj, 1]], ssem.at[p], add=True)

    @pl.when(j + 2 < NCH)
    def _prefetch_cidx():
      cidx_copy(j + 2, q2)

    return 0

  lax.fori_loop(0, NCH, chunk, 0)
  pltpu.make_async_copy(
      gbuf.at[(NCH - 1) & 1], table.at[cbuf.at[(NCH - 1) & 3, 1]],
      ssem.at[(NCH - 1) & 1]).wait()

  pltpu.sync_copy(hist, shacc.at[idx80], add=True)
  plsc.subcore_barrier()

  @pl.when(s < HR // HRPT)
  def _read_shacc():
    pltpu.sync_copy(shacc.at[pl.ds(hbase, HRPT)], rbuf)
    pltpu.sync_copy(rbuf, deg_hbm.at[c, pl.ds(hbase, HRPT)])

  for off, sz in _ROW_CHUNKS:
    pltpu.sync_copy(table.at[pl.ds(base + off, sz)], gbuf.at[0, pl.ds(0, sz)])
    pltpu.sync_copy(gbuf.at[0, pl.ds(0, sz)],
                    agg_hbm.at[c, pl.ds(base + off, sz)])


@jax.jit
def _sc_segment_sums(y_pad, pk):
  mesh = plsc.VectorSubcoreMesh(
      core_axis_name="c", subcore_axis_name="s", num_cores=NC, num_subcores=NS)
  return pl.kernel(
      _sc_body,
      out_type=(
          jax.ShapeDtypeStruct((NC, N_T, D), jnp.float32),
          jax.ShapeDtypeStruct((NC, HR, D), jnp.float32),
      ),
      mesh=mesh,
      compiler_params=pltpu.CompilerParams(needs_layout_passes=False),
      scratch_types=[
          pltpu.VMEM((4, 8, C), jnp.int32),
          pltpu.VMEM((2, C, D), jnp.float32),
          pltpu.VMEM((HR, D), jnp.float32),
          pltpu.VMEM((HRPT, D), jnp.float32),
          pltpu.VMEM((HR,), jnp.int32),
          pltpu.VMEM_SHARED((N_T, D), jnp.float32),
          pltpu.VMEM_SHARED((HR, D), jnp.float32),
          pltpu.SemaphoreType.DMA((2,)),
          pltpu.SemaphoreType.DMA((2,)),
      ],
  )(y_pad, pk)


def _tc_body(y_ref, x_ref, agg_ref, deg_ref, hr_ref, hri_ref,
             alp_ref, lam_ref, o_ref):
  y = y_ref[...]
  x = x_ref[...]
  s1 = agg_ref[0]
  s2 = agg_ref[1]
  d_ri = deg_ref[0]
  d_r = deg_ref[1]
  hr = hr_ref[...]
  hri = hri_ref[...]
  alp = alp_ref[0, 0]
  lam = lam_ref[0, 0]

  dot = functools.partial(
      lax.dot_general, dimension_numbers=(((1,), (0,)), ((), ())),
      preferred_element_type=jnp.float32)
  dott = functools.partial(
      lax.dot_general, dimension_numbers=(((1,), (1,)), ((), ())),
      preferred_element_type=jnp.float32)

  yhr = dott(dot(y, hr), hr)
  yhri = dott(dot(y, hri), hri)
  a1 = dot(s1, hr) + dott(s1, hri)
  a2 = dot(s2, hri) + dott(s2, hr)
  deg = d_r + d_ri

  r = x + a1 + a2 - d_r * yhr - d_ri * yhri
  r = (1.0 - alp) * y + (alp * lam) * r / (1.0 + lam * deg)
  o_ref[...] = jnp.maximum(r, 0.0)


@jax.jit
def _tc_combine(y, x, agg, deg, hr, hri, alp, lam):
  blk = 1000
  grid = N // blk
  return pl.pallas_call(
      _tc_body,
      grid=(grid,),
      in_specs=[
          pl.BlockSpec((blk, D), lambda i: (i, 0)),
          pl.BlockSpec((blk, D), lambda i: (i, 0)),
          pl.BlockSpec((NC, blk, D), lambda i: (0, i, 0)),
          pl.BlockSpec((NC, blk, 1), lambda i: (0, i, 0)),
          pl.BlockSpec((D, D), lambda i: (0, 0)),
          pl.BlockSpec((D, D), lambda i: (0, 0)),
          pl.BlockSpec(memory_space=pltpu.SMEM),
          pl.BlockSpec(memory_space=pltpu.SMEM),
      ],
      out_specs=pl.BlockSpec((blk, D), lambda i: (i, 0)),
      out_shape=jax.ShapeDtypeStruct((N, D), jnp.float32),
  )(y, x, agg, deg, hr, hri, alp, lam)


def kernel(Y, X, edge_index, w_r, w_rinv, H_r, H_rinv, alp, lam):
  src = edge_index[0]
  dst = edge_index[1]
  npad = E_PAD - E
  pad_idx = jnp.full((npad,), N, dtype=jnp.int32)
  pad_w = jnp.zeros((npad,), dtype=jnp.float32)

  src_p = jnp.concatenate([src, pad_idx])
  dst_p = jnp.concatenate([dst, pad_idx])
  wr_b = lax.bitcast_convert_type(
      jnp.concatenate([w_r[:, 0], pad_w]), jnp.int32)
  wri_b = lax.bitcast_convert_type(
      jnp.concatenate([w_rinv[:, 0], pad_w]), jnp.int32)

  shape4 = (NC, NS, NCH, C)
  gidx = jnp.stack([src_p, dst_p]).reshape(shape4)
  sidx = jnp.stack([dst_p, src_p]).reshape(shape4)
  wb = jnp.stack([wr_b, wri_b]).reshape(shape4)
  pk = jnp.stack([gidx, sidx, wb], axis=3)
  y_pad = jnp.concatenate(
      [Y, jnp.zeros((N_T - N, D), dtype=jnp.float32)], axis=0)

  agg, deg = _sc_segment_sums(y_pad, pk)
  deg3 = deg.reshape(NC, HR * D)[:, :N].reshape(NC, N, 1)
  alp11 = jnp.reshape(alp, (1, 1)).astype(jnp.float32)
  lam11 = jnp.reshape(lam, (1, 1)).astype(jnp.float32)
  return _tc_combine(Y, X, agg[:, :N], deg3, H_r, H_rinv, alp11, lam11)


# speedup vs baseline: 1.2211x; 1.2211x over previous
import functools

import jax
import jax.numpy as jnp
from jax import lax
from jax.experimental import pallas as pl
from jax.experimental.pallas import tpu as pltpu
from jax.experimental.pallas import tpu_sc as plsc

N = 10000
E = 320000
D = 128
NC = 2
NS = 16
L = 16
C = 128
NCH = 160
SCH = 8
EPT = NCH * C
E_PAD = NS * EPT
N_T = 10112
RPT = N_T // NS
HR = 80
HRPT = 8
_ROW_CHUNKS = ((0, 128), (128, 128), (256, 128), (384, 128), (512, 120))


def _sc_body(y_hbm, pk_hbm, agg_hbm, deg_hbm,
             cbuf, gbuf, hist, rbuf, idx80, table, shacc, gsem):
  c = lax.axis_index("c")
  s = lax.axis_index("s")

  zeros16 = jnp.zeros((L,), jnp.float32)
  ones16 = jnp.ones((L,), jnp.float32)
  lanes = lax.iota(jnp.int32, L)

  def zhist(i, _):
    for g in range(D // L):
      hist[i, pl.ds(g * L, L)] = zeros16
    return 0

  lax.fori_loop(0, HR, zhist, 0)

  for i in range(HRPT):
    for g in range(D // L):
      rbuf[i, pl.ds(g * L, L)] = zeros16
  for k in range(HR // L):
    idx80[pl.ds(k * L, L)] = lanes + (k * L)

  hbase = s * HRPT

  @pl.when(s < HR // HRPT)
  def _zero_shacc():
    pltpu.sync_copy(rbuf, shacc.at[pl.ds(hbase, HRPT)])

  def zrow(i, _):
    for g in range(D // L):
      gbuf[i, pl.ds(g * L, L)] = zeros16
    return 0

  lax.fori_loop(0, C, zrow, 0)
  base = s * RPT
  for off, sz in _ROW_CHUNKS:
    pltpu.sync_copy(gbuf.at[pl.ds(0, sz)], table.at[pl.ds(base + off, sz)])

  plsc.subcore_barrier()

  def chunk(j, _):
    pltpu.sync_copy(pk_hbm.at[c, s, j], cbuf)
    pltpu.async_copy(y_hbm.at[cbuf.at[0]], gbuf, gsem).wait()

    def group(g, _):
      wv = plsc.bitcast(cbuf[2, pl.ds(g * L, L)], jnp.float32)
      iv = cbuf[1, pl.ds(g * L, L)]
      ir = lax.shift_right_logical(iv, 7)
      ic = lax.bitwise_and(iv, jnp.int32(D - 1))
      for k in range(L):
        i = g * L + k
        w = wv[k]
        for gg in range(D // L):
          gbuf[i, pl.ds(gg * L, L)] = gbuf[i, pl.ds(gg * L, L)] * w
      for k in range(L):
        plsc.addupdate_scatter(hist, [ir, ic], ones16, mask=lanes == k)
      return 0

    lax.fori_loop(0, C // L, group, 0)
    pltpu.sync_copy(gbuf, table.at[cbuf.at[1]], add=True)
    return 0

  lax.fori_loop(0, NCH, chunk, 0)

  pltpu.sync_copy(hist, shacc.at[idx80], add=True)
  plsc.subcore_barrier()

  @pl.when(s < HR // HRPT)
  def _read_shacc():
    pltpu.sync_copy(shacc.at[pl.ds(hbase, HRPT)], rbuf)
    pltpu.sync_copy(rbuf, deg_hbm.at[c, pl.ds(hbase, HRPT)])

  for off, sz in _ROW_CHUNKS:
    pltpu.sync_copy(table.at[pl.ds(base + off, sz)], gbuf.at[pl.ds(0, sz)])
    pltpu.sync_copy(gbuf.at[pl.ds(0, sz)],
                    agg_hbm.at[c, pl.ds(base + off, sz)])


@jax.jit
def _sc_segment_sums(y_pad, pk):
  mesh = plsc.VectorSubcoreMesh(
      core_axis_name="c", subcore_axis_name="s", num_cores=NC, num_subcores=NS)
  return pl.kernel(
      _sc_body,
      out_type=(
          jax.ShapeDtypeStruct((NC, N_T, D), jnp.float32),
          jax.ShapeDtypeStruct((NC, HR, D), jnp.float32),
      ),
      mesh=mesh,
      compiler_params=pltpu.CompilerParams(needs_layout_passes=False),
      scratch_types=[
          pltpu.VMEM((3, C), jnp.int32),
          pltpu.VMEM((C, D), jnp.float32),
          pltpu.VMEM((HR, D), jnp.float32),
          pltpu.VMEM((HRPT, D), jnp.float32),
          pltpu.VMEM((HR,), jnp.int32),
          pltpu.VMEM_SHARED((N_T, D), jnp.float32),
          pltpu.VMEM_SHARED((HR, D), jnp.float32),
          pltpu.SemaphoreType.DMA,
      ],
  )(y_pad, pk)


def _tc_body(y_ref, x_ref, agg_ref, deg_ref, hr_ref, hri_ref,
             alp_ref, lam_ref, o_ref):
  y = y_ref[...]
  x = x_ref[...]
  s1 = agg_ref[0]
  s2 = agg_ref[1]
  d_ri = deg_ref[0]
  d_r = deg_ref[1]
  hr = hr_ref[...]
  hri = hri_ref[...]
  alp = alp_ref[0, 0]
  lam = lam_ref[0, 0]

  dot = functools.partial(
      lax.dot_general, dimension_numbers=(((1,), (0,)), ((), ())),
      preferred_element_type=jnp.float32)
  dott = functools.partial(
      lax.dot_general, dimension_numbers=(((1,), (1,)), ((), ())),
      preferred_element_type=jnp.float32)

  yhr = dott(dot(y, hr), hr)
  yhri = dott(dot(y, hri), hri)
  a1 = dot(s1, hr) + dott(s1, hri)
  a2 = dot(s2, hri) + dott(s2, hr)
  deg = d_r + d_ri

  r = x + a1 + a2 - d_r * yhr - d_ri * yhri
  r = (1.0 - alp) * y + (alp * lam) * r / (1.0 + lam * deg)
  o_ref[...] = jnp.maximum(r, 0.0)


@jax.jit
def _tc_combine(y, x, agg, deg, hr, hri, alp, lam):
  blk = 1000
  grid = N // blk
  return pl.pallas_call(
      _tc_body,
      grid=(grid,),
      in_specs=[
          pl.BlockSpec((blk, D), lambda i: (i, 0)),
          pl.BlockSpec((blk, D), lambda i: (i, 0)),
          pl.BlockSpec((NC, blk, D), lambda i: (0, i, 0)),
          pl.BlockSpec((NC, blk, 1), lambda i: (0, i, 0)),
          pl.BlockSpec((D, D), lambda i: (0, 0)),
          pl.BlockSpec((D, D), lambda i: (0, 0)),
          pl.BlockSpec(memory_space=pltpu.SMEM),
          pl.BlockSpec(memory_space=pltpu.SMEM),
      ],
      out_specs=pl.BlockSpec((blk, D), lambda i: (i, 0)),
      out_shape=jax.ShapeDtypeStruct((N, D), jnp.float32),
  )(y, x, agg, deg, hr, hri, alp, lam)


def kernel(Y, X, edge_index, w_r, w_rinv, H_r, H_rinv, alp, lam):
  src = edge_index[0]
  dst = edge_index[1]
  npad = E_PAD - E
  pad_idx = jnp.full((npad,), N, dtype=jnp.int32)
  pad_w = jnp.zeros((npad,), dtype=jnp.float32)

  src_p = jnp.concatenate([src, pad_idx])
  dst_p = jnp.concatenate([dst, pad_idx])
  wr_b = lax.bitcast_convert_type(
      jnp.concatenate([w_r[:, 0], pad_w]), jnp.int32)
  wri_b = lax.bitcast_convert_type(
      jnp.concatenate([w_rinv[:, 0], pad_w]), jnp.int32)

  shape4 = (NC, NS, NCH, C)
  gidx = jnp.stack([src_p, dst_p]).reshape(shape4)
  sidx = jnp.stack([dst_p, src_p]).reshape(shape4)
  wb = jnp.stack([wr_b, wri_b]).reshape(shape4)
  pk = jnp.stack([gidx, sidx, wb], axis=3)
  y_pad = jnp.concatenate(
      [Y, jnp.zeros((N_T - N, D), dtype=jnp.float32)], axis=0)

  agg, deg = _sc_segment_sums(y_pad, pk)
  deg3 = deg.reshape(NC, HR * D)[:, :N].reshape(NC, N, 1)
  alp11 = jnp.reshape(alp, (1, 1)).astype(jnp.float32)
  lam11 = jnp.reshape(lam, (1, 1)).astype(jnp.float32)
  return _tc_combine(Y, X, agg[:, :N], deg3, H_r, H_rinv, alp11, lam11)


# speedup vs baseline: 2.0490x; 1.6779x over previous
import functools

import jax
import jax.numpy as jnp
from jax import lax
from jax.experimental import pallas as pl
from jax.experimental.pallas import tpu as pltpu
from jax.experimental.pallas import tpu_sc as plsc

N = 10000
E = 320000
D = 128
NC = 2
NS = 16
L = 16
C = 128
NCH = 160
SCH = 8
EPT = NCH * C
E_PAD = NS * EPT
N_T = 10112
RPT = N_T // NS
HR = 80
HRPT = 8
_ROW_CHUNKS = ((0, 128), (128, 128), (256, 128), (384, 128), (512, 120))


def _sc_body(y_hbm, pk_hbm, agg_hbm, deg_hbm,
             cbuf, gbuf, hist, rbuf, idx80, table, shacc, gsem):
  c = lax.axis_index("c")
  s = lax.axis_index("s")

  zeros16 = jnp.zeros((L,), jnp.float32)
  ones16 = jnp.ones((L,), jnp.float32)
  lanes = lax.iota(jnp.int32, L)

  def zhist(i, _):
    for g in range(D // L):
      hist[i, pl.ds(g * L, L)] = zeros16
    return 0

  lax.fori_loop(0, HR, zhist, 0)

  for i in range(HRPT):
    for g in range(D // L):
      rbuf[i, pl.ds(g * L, L)] = zeros16
  for k in range(HR // L):
    idx80[pl.ds(k * L, L)] = lanes + (k * L)

  hbase = s * HRPT

  @pl.when(s < HR // HRPT)
  def _zero_shacc():
    pltpu.sync_copy(rbuf, shacc.at[pl.ds(hbase, HRPT)])

  def zrow(i, _):
    for g in range(D // L):
      gbuf[i, pl.ds(g * L, L)] = zeros16
    return 0

  lax.fori_loop(0, C, zrow, 0)
  base = s * RPT
  for off, sz in _ROW_CHUNKS:
    pltpu.sync_copy(gbuf.at[pl.ds(0, sz)], table.at[pl.ds(base + off, sz)])

  plsc.subcore_barrier()

  def chunk(j, _):
    pltpu.sync_copy(pk_hbm.at[c, s, j], cbuf)
    pltpu.async_copy(y_hbm.at[cbuf.at[0]], gbuf, gsem).wait()

    def group(g, _):
      wv = plsc.bitcast(cbuf[2, pl.ds(g * L, L)], jnp.float32)
      iv = cbuf[1, pl.ds(g * L, L)]
      ir = lax.shift_right_logical(iv, 7)
      ic = lax.bitwise_and(iv, jnp.int32(D - 1))
      for k in range(L):
        i = g * L + k
        w = wv[k]
        for gg in range(D // L):
          gbuf[i, pl.ds(gg * L, L)] = gbuf[i, pl.ds(gg * L, L)] * w
      for k in range(L):
        plsc.addupdate_scatter(hist, [ir, ic], ones16, mask=lanes == k)
      return 0

    lax.fori_loop(0, C // L, group, 0)
    pltpu.sync_copy(gbuf, table.at[cbuf.at[1]], add=True)
    return 0

  lax.fori_loop(0, NCH, chunk, 0)

  pltpu.sync_copy(hist, shacc.at[idx80], add=True)
  plsc.subcore_barrier()

  @pl.when(s < HR // HRPT)
  def _read_shacc():
    pltpu.sync_copy(shacc.at[pl.ds(hbase, HRPT)], rbuf)
    pltpu.sync_copy(rbuf, deg_hbm.at[c, pl.ds(hbase, HRPT)])

  for off, sz in _ROW_CHUNKS:
    pltpu.sync_copy(table.at[pl.ds(base + off, sz)], gbuf.at[pl.ds(0, sz)])
    pltpu.sync_copy(gbuf.at[pl.ds(0, sz)],
                    agg_hbm.at[c, pl.ds(base + off, sz)])


@jax.jit
def _sc_segment_sums(y_pad, pk):
  mesh = plsc.VectorSubcoreMesh(
      core_axis_name="c", subcore_axis_name="s", num_cores=NC, num_subcores=NS)
  return pl.kernel(
      _sc_body,
      out_type=(
          jax.ShapeDtypeStruct((NC, N_T, D), jnp.float32),
          jax.ShapeDtypeStruct((NC, HR, D), jnp.float32),
      ),
      mesh=mesh,
      compiler_params=pltpu.CompilerParams(needs_layout_passes=False),
      scratch_types=[
          pltpu.VMEM((3, C), jnp.int32),
          pltpu.VMEM((C, D), jnp.float32),
          pltpu.VMEM((HR, D), jnp.float32),
          pltpu.VMEM((HRPT, D), jnp.float32),
          pltpu.VMEM((HR,), jnp.int32),
          pltpu.VMEM_SHARED((N_T, D), jnp.float32),
          pltpu.VMEM_SHARED((HR, D), jnp.float32),
          pltpu.SemaphoreType.DMA,
      ],
  )(y_pad, pk)


def _tc_body(y_ref, x_ref, agg_ref, deg_ref, hr_ref, hri_ref,
             alp_ref, lam_ref, o_ref):
  y = y_ref[...]
  x = x_ref[...]
  s1 = agg_ref[0]
  s2 = agg_ref[1]
  d_ri = deg_ref[0]
  d_r = deg_ref[1]
  hr = hr_ref[...]
  hri = hri_ref[...]
  alp = alp_ref[0, 0]
  lam = lam_ref[0, 0]

  dot = functools.partial(
      lax.dot_general, dimension_numbers=(((1,), (0,)), ((), ())),
      preferred_element_type=jnp.float32)
  dott = functools.partial(
      lax.dot_general, dimension_numbers=(((1,), (1,)), ((), ())),
      preferred_element_type=jnp.float32)

  yhr = dott(dot(y, hr), hr)
  yhri = dott(dot(y, hri), hri)
  a1 = dot(s1, hr) + dott(s1, hri)
  a2 = dot(s2, hri) + dott(s2, hr)
  deg = d_r + d_ri

  r = x + a1 + a2 - d_r * yhr - d_ri * yhri
  r = (1.0 - alp) * y + (alp * lam) * r / (1.0 + lam * deg)
  o_ref[...] = jnp.maximum(r, 0.0)


@jax.jit
def _tc_combine(y, x, agg, deg, hr, hri, alp, lam):
  blk = 1000
  grid = N // blk
  return pl.pallas_call(
      _tc_body,
      grid=(grid,),
      in_specs=[
          pl.BlockSpec((blk, D), lambda i: (i, 0)),
          pl.BlockSpec((blk, D), lambda i: (i, 0)),
          pl.BlockSpec((NC, blk, D), lambda i: (0, i, 0)),
          pl.BlockSpec((NC, blk, 1), lambda i: (0, i, 0)),
          pl.BlockSpec((D, D), lambda i: (0, 0)),
          pl.BlockSpec((D, D), lambda i: (0, 0)),
          pl.BlockSpec(memory_space=pltpu.SMEM),
          pl.BlockSpec(memory_space=pltpu.SMEM),
      ],
      out_specs=pl.BlockSpec((blk, D), lambda i: (i, 0)),
      out_shape=jax.ShapeDtypeStruct((N, D), jnp.float32),
  )(y, x, agg, deg, hr, hri, alp, lam)


def kernel(Y, X, edge_index, w_r, w_rinv, H_r, H_rinv, alp, lam):
  src = edge_index[0]
  dst = edge_index[1]
  npad = E_PAD - E
  pad_idx = N + (jnp.arange(npad, dtype=jnp.int32) % (N_T - N))
  pad_w = jnp.zeros((npad,), dtype=jnp.float32)

  src_p = jnp.concatenate([src, pad_idx])
  dst_p = jnp.concatenate([dst, pad_idx])
  wr_b = lax.bitcast_convert_type(
      jnp.concatenate([w_r[:, 0], pad_w]), jnp.int32)
  wri_b = lax.bitcast_convert_type(
      jnp.concatenate([w_rinv[:, 0], pad_w]), jnp.int32)

  shape4 = (NC, NS, NCH, C)
  gidx = jnp.stack([src_p, dst_p]).reshape(shape4)
  sidx = jnp.stack([dst_p, src_p]).reshape(shape4)
  wb = jnp.stack([wr_b, wri_b]).reshape(shape4)
  pk = jnp.stack([gidx, sidx, wb], axis=3)
  y_pad = jnp.concatenate(
      [Y, jnp.zeros((N_T - N, D), dtype=jnp.float32)], axis=0)

  agg, deg = _sc_segment_sums(y_pad, pk)
  deg3 = deg.reshape(NC, HR * D)[:, :N].reshape(NC, N, 1)
  alp11 = jnp.reshape(alp, (1, 1)).astype(jnp.float32)
  lam11 = jnp.reshape(lam, (1, 1)).astype(jnp.float32)
  return _tc_combine(Y, X, agg[:, :N], deg3, H_r, H_rinv, alp11, lam11)


# speedup vs baseline: 2.0698x; 1.0102x over previous
import functools

import jax
import jax.numpy as jnp
from jax import lax
from jax.experimental import pallas as pl
from jax.experimental.pallas import tpu as pltpu
from jax.experimental.pallas import tpu_sc as plsc

N = 10000
E = 320000
D = 128
NC = 2
NS = 16
L = 16
C = 128
NCH = 160
SCH = 8
EPT = NCH * C
E_PAD = NS * EPT
N_T = 10112
RPT = N_T // NS
HR = 80
HRPT = 8
_ROW_CHUNKS = ((0, 128), (128, 128), (256, 128), (384, 128), (512, 120))


def _sc_body(y_hbm, pk_hbm, agg_hbm, deg_hbm,
             cbuf, gbuf, hist, rbuf, idx80, table, shacc, gsem):
  c = lax.axis_index("c")
  s = lax.axis_index("s")

  zeros16 = jnp.zeros((L,), jnp.float32)
  ones16 = jnp.ones((L,), jnp.float32)
  lanes = lax.iota(jnp.int32, L)

  def zhist(i, _):
    for g in range(D // L):
      hist[i, pl.ds(g * L, L)] = zeros16
    return 0

  lax.fori_loop(0, HR, zhist, 0)

  for i in range(HRPT):
    for g in range(D // L):
      rbuf[i, pl.ds(g * L, L)] = zeros16
  for k in range(HR // L):
    idx80[pl.ds(k * L, L)] = lanes + (k * L)

  hbase = s * HRPT

  @pl.when(s < HR // HRPT)
  def _zero_shacc():
    pltpu.sync_copy(rbuf, shacc.at[pl.ds(hbase, HRPT)])

  def zrow(i, _):
    for g in range(D // L):
      gbuf[i, pl.ds(g * L, L)] = zeros16
    return 0

  lax.fori_loop(0, C, zrow, 0)
  base = s * RPT
  for off, sz in _ROW_CHUNKS:
    pltpu.sync_copy(gbuf.at[pl.ds(0, sz)], table.at[pl.ds(base + off, sz)])

  plsc.subcore_barrier()

  def chunk(j, _):
    pltpu.sync_copy(pk_hbm.at[c, s, j], cbuf)
    pltpu.async_copy(y_hbm.at[cbuf.at[0]], gbuf, gsem).wait()

    def group(g, _):
      wv = plsc.bitcast(cbuf[2, pl.ds(g * L, L)], jnp.float32)
      iv = cbuf[1, pl.ds(g * L, L)]
      ir = lax.shift_right_logical(iv, 7)
      ic = lax.bitwise_and(iv, jnp.int32(D - 1))
      for k in range(L):
        i = g * L + k
        w = wv[k]
        for gg in range(D // L):
          gbuf[i, pl.ds(gg * L, L)] = gbuf[i, pl.ds(gg * L, L)] * w
      for k in range(L):
        plsc.addupdate_scatter(hist, [ir, ic], ones16, mask=lanes == k)
      return 0

    lax.fori_loop(0, C // L, group, 0)
    pltpu.sync_copy(gbuf, table.at[cbuf.at[1]], add=True)
    return 0

  lax.fori_loop(0, NCH, chunk, 0)

  pltpu.sync_copy(hist, shacc.at[idx80], add=True)
  plsc.subcore_barrier()

  @pl.when(s < HR // HRPT)
  def _read_shacc():
    pltpu.sync_copy(shacc.at[pl.ds(hbase, HRPT)], rbuf)
    pltpu.sync_copy(rbuf, deg_hbm.at[c, pl.ds(hbase, HRPT)])

  for off, sz in _ROW_CHUNKS:
    pltpu.sync_copy(table.at[pl.ds(base + off, sz)], gbuf.at[pl.ds(0, sz)])
    pltpu.sync_copy(gbuf.at[pl.ds(0, sz)],
                    agg_hbm.at[c, pl.ds(base + off, sz)])


@jax.jit
def _sc_segment_sums(y_pad, pk):
  mesh = plsc.VectorSubcoreMesh(
      core_axis_name="c", subcore_axis_name="s", num_cores=NC, num_subcores=NS)
  return pl.kernel(
      _sc_body,
      out_type=(
          jax.ShapeDtypeStruct((NC, N_T, D), jnp.float32),
          jax.ShapeDtypeStruct((NC, HR, D), jnp.float32),
      ),
      mesh=mesh,
      compiler_params=pltpu.CompilerParams(needs_layout_passes=False),
      scratch_types=[
          pltpu.VMEM((3, C), jnp.int32),
          pltpu.VMEM((C, D), jnp.float32),
          pltpu.VMEM((HR, D), jnp.float32),
          pltpu.VMEM((HRPT, D), jnp.float32),
          pltpu.VMEM((HR,), jnp.int32),
          pltpu.VMEM_SHARED((N_T, D), jnp.float32),
          pltpu.VMEM_SHARED((HR, D), jnp.float32),
          pltpu.SemaphoreType.DMA,
      ],
  )(y_pad, pk)


def _tc_body(y_ref, x_ref, agg_ref, deg_ref, hr_ref, hri_ref,
             alp_ref, lam_ref, o_ref):
  y = y_ref[...]
  x = x_ref[...]
  s1 = agg_ref[0]
  s2 = agg_ref[1]
  d_ri = deg_ref[0]
  d_r = deg_ref[1]
  hr = hr_ref[...]
  hri = hri_ref[...]
  alp = alp_ref[0, 0]
  lam = lam_ref[0, 0]

  dot = functools.partial(
      lax.dot_general, dimension_numbers=(((1,), (0,)), ((), ())),
      preferred_element_type=jnp.float32)
  dott = functools.partial(
      lax.dot_general, dimension_numbers=(((1,), (1,)), ((), ())),
      preferred_element_type=jnp.float32)

  yhr = dott(dot(y, hr), hr)
  yhri = dott(dot(y, hri), hri)
  a1 = dot(s1, hr) + dott(s1, hri)
  a2 = dot(s2, hri) + dott(s2, hr)
  deg = d_r + d_ri

  r = x + a1 + a2 - d_r * yhr - d_ri * yhri
  r = (1.0 - alp) * y + (alp * lam) * r / (1.0 + lam * deg)
  o_ref[...] = jnp.maximum(r, 0.0)


@jax.jit
def _tc_combine(y, x, agg, deg, hr, hri, alp, lam):
  blk = 1000
  grid = N // blk
  return pl.pallas_call(
      _tc_body,
      grid=(grid,),
      in_specs=[
          pl.BlockSpec((blk, D), lambda i: (i, 0)),
          pl.BlockSpec((blk, D), lambda i: (i, 0)),
          pl.BlockSpec((NC, blk, D), lambda i: (0, i, 0)),
          pl.BlockSpec((NC, blk, 1), lambda i: (0, i, 0)),
          pl.BlockSpec((D, D), lambda i: (0, 0)),
          pl.BlockSpec((D, D), lambda i: (0, 0)),
          pl.BlockSpec(memory_space=pltpu.SMEM),
          pl.BlockSpec(memory_space=pltpu.SMEM),
      ],
      out_specs=pl.BlockSpec((blk, D), lambda i: (i, 0)),
      out_shape=jax.ShapeDtypeStruct((N, D), jnp.float32),
  )(y, x, agg, deg, hr, hri, alp, lam)


def kernel(Y, X, edge_index, w_r, w_rinv, H_r, H_rinv, alp, lam):
  src = edge_index[0]
  dst = edge_index[1]
  npad = E_PAD - E
  cyc = jnp.arange(npad, dtype=jnp.int32) % (N_T - N)
  pad_g = cyc
  pad_s = N + cyc
  pad_w = jnp.zeros((npad,), dtype=jnp.float32)

  src_g = jnp.concatenate([src, pad_g])
  dst_g = jnp.concatenate([dst, pad_g])
  src_s = jnp.concatenate([src, pad_s])
  dst_s = jnp.concatenate([dst, pad_s])
  wr_b = lax.bitcast_convert_type(
      jnp.concatenate([w_r[:, 0], pad_w]), jnp.int32)
  wri_b = lax.bitcast_convert_type(
      jnp.concatenate([w_rinv[:, 0], pad_w]), jnp.int32)

  shape4 = (NC, NS, NCH, C)
  gidx = jnp.stack([src_g, dst_g]).reshape(shape4)
  sidx = jnp.stack([dst_s, src_s]).reshape(shape4)
  wb = jnp.stack([wr_b, wri_b]).reshape(shape4)
  pk = jnp.stack([gidx, sidx, wb], axis=3)

  agg, deg = _sc_segment_sums(Y, pk)
  deg3 = deg.reshape(NC, HR * D, 1)
  alp11 = jnp.reshape(alp, (1, 1)).astype(jnp.float32)
  lam11 = jnp.reshape(lam, (1, 1)).astype(jnp.float32)
  return _tc_combine(Y, X, agg, deg3, H_r, H_rinv, alp11, lam11)


# speedup vs baseline: 2.8676x; 1.3854x over previous
import functools

import jax
import jax.numpy as jnp
from jax import lax
from jax.experimental import pallas as pl
from jax.experimental.pallas import tpu as pltpu
from jax.experimental.pallas import tpu_sc as plsc

N = 10000
E = 320000
D = 128
NC = 2
NS = 16
L = 16
C = 128
NCH = 160
SCH = 8
EPT = NCH * C
E_PAD = NS * EPT
N_T = 10112
RPT = N_T // NS
HR = 80
HRPT = 8
_ROW_CHUNKS = ((0, 128), (128, 128), (256, 128), (384, 128), (512, 120))


def _sc_body(y_hbm, pk_hbm, agg_hbm, deg_hbm,
             cbuf0, cbuf1, cbuf2, cbuf3, gbufa, gbufb,
             hist, rbuf, idx80, table, shacc, csem, gsem, ssem):
  c = lax.axis_index("c")
  s = lax.axis_index("s")

  zeros16 = jnp.zeros((L,), jnp.float32)
  ones16 = jnp.ones((L,), jnp.float32)
  lanes = lax.iota(jnp.int32, L)

  def zhist(i, _):
    for g in range(D // L):
      hist[i, pl.ds(g * L, L)] = zeros16
    return 0

  lax.fori_loop(0, HR, zhist, 0)

  for i in range(HRPT):
    for g in range(D // L):
      rbuf[i, pl.ds(g * L, L)] = zeros16
  for k in range(HR // L):
    idx80[pl.ds(k * L, L)] = lanes + (k * L)

  hbase = s * HRPT

  @pl.when(s < HR // HRPT)
  def _zero_shacc():
    pltpu.sync_copy(rbuf, shacc.at[pl.ds(hbase, HRPT)])

  def zrow(i, _):
    for g in range(D // L):
      gbufa[i, pl.ds(g * L, L)] = zeros16
    return 0

  lax.fori_loop(0, C, zrow, 0)
  base = s * RPT
  for off, sz in _ROW_CHUNKS:
    pltpu.sync_copy(gbufa.at[pl.ds(0, sz)], table.at[pl.ds(base + off, sz)])

  plsc.subcore_barrier()

  cbufs = (cbuf0, cbuf1, cbuf2, cbuf3)
  gbufs = (gbufa, gbufb)

  def start_c(j, q):
    pltpu.async_copy(pk_hbm.at[c, s, j], cbufs[q], csem.at[q])

  def wait_c(j, q):
    pltpu.make_async_copy(pk_hbm.at[c, s, j], cbufs[q], csem.at[q]).wait()

  def start_g(q, p):
    pltpu.async_copy(y_hbm.at[cbufs[q].at[0]], gbufs[p], gsem.at[p])

  def wait_g(q, p):
    pltpu.make_async_copy(y_hbm.at[cbufs[q].at[0]], gbufs[p], gsem.at[p]).wait()

  def start_s(q, p):
    pltpu.async_copy(gbufs[p], table.at[cbufs[q].at[1]], ssem.at[p], add=True)

  def wait_s(q, p):
    pltpu.make_async_copy(gbufs[p], table.at[cbufs[q].at[1]], ssem.at[p]).wait()

  def compute(cb, gb):
    def group(g, _):
      wv = plsc.bitcast(cb[2, pl.ds(g * L, L)], jnp.float32)
      iv = cb[1, pl.ds(g * L, L)]
      ir = lax.shift_right_logical(iv, 7)
      ic = lax.bitwise_and(iv, jnp.int32(D - 1))
      for k in range(L):
        i = g * L + k
        w = wv[k]
        for gg in range(D // L):
          gb[i, pl.ds(gg * L, L)] = gb[i, pl.ds(gg * L, L)] * w
      for k in range(L):
        plsc.addupdate_scatter(hist, [ir, ic], ones16, mask=lanes == k)
      return 0

    lax.fori_loop(0, C // L, group, 0)

  def step(j, b, first=False, prefetch_c=True, prefetch_g=True):
    p = b & 1
    pn = p ^ 1
    wait_g(b, p)
    compute(cbufs[b], gbufs[p])
    start_s(b, p)
    if not first:
      wait_s((b - 1) & 3, pn)
    if prefetch_c:
      start_c(j + 3, (b + 3) & 3)
    if prefetch_g:
      wait_c(j + 1, (b + 1) & 3)
      start_g((b + 1) & 3, pn)

  start_c(0, 0)
  start_c(1, 1)
  start_c(2, 2)
  wait_c(0, 0)
  start_g(0, 0)
  for b in range(4):
    step(b, b, first=(b == 0))

  def quad(jj, _):
    for b in range(4):
      step(jj * 4 + b, b)
    return 0

  lax.fori_loop(1, NCH // 4 - 1, quad, 0)

  for b in range(4):
    j = NCH - 4 + b
    step(j, b, prefetch_c=(j + 3 < NCH), prefetch_g=(j + 1 < NCH))
  wait_s(3, 1)

  pltpu.sync_copy(hist, shacc.at[idx80], add=True)
  plsc.subcore_barrier()

  @pl.when(s < HR // HRPT)
  def _read_shacc():
    pltpu.sync_copy(shacc.at[pl.ds(hbase, HRPT)], rbuf)
    pltpu.sync_copy(rbuf, deg_hbm.at[c, pl.ds(hbase, HRPT)])

  for off, sz in _ROW_CHUNKS:
    pltpu.sync_copy(table.at[pl.ds(base + off, sz)], gbufa.at[pl.ds(0, sz)])
    pltpu.sync_copy(gbufa.at[pl.ds(0, sz)],
                    agg_hbm.at[c, pl.ds(base + off, sz)])


@jax.jit
def _sc_segment_sums(y_pad, pk):
  mesh = plsc.VectorSubcoreMesh(
      core_axis_name="c", subcore_axis_name="s", num_cores=NC, num_subcores=NS)
  return pl.kernel(
      _sc_body,
      out_type=(
          jax.ShapeDtypeStruct((NC, N_T, D), jnp.float32),
          jax.ShapeDtypeStruct((NC, HR, D), jnp.float32),
      ),
      mesh=mesh,
      compiler_params=pltpu.CompilerParams(needs_layout_passes=False),
      scratch_types=[
          pltpu.VMEM((3, C), jnp.int32),
          pltpu.VMEM((3, C), jnp.int32),
          pltpu.VMEM((3, C), jnp.int32),
          pltpu.VMEM((3, C), jnp.int32),
          pltpu.VMEM((C, D), jnp.float32),
          pltpu.VMEM((C, D), jnp.float32),
          pltpu.VMEM((HR, D), jnp.float32),
          pltpu.VMEM((HRPT, D), jnp.float32),
          pltpu.VMEM((HR,), jnp.int32),
          pltpu.VMEM_SHARED((N_T, D), jnp.float32),
          pltpu.VMEM_SHARED((HR, D), jnp.float32),
          pltpu.SemaphoreType.DMA((4,)),
          pltpu.SemaphoreType.DMA((2,)),
          pltpu.SemaphoreType.DMA((2,)),
      ],
  )(y_pad, pk)


def _tc_body(y_ref, x_ref, agg_ref, deg_ref, hr_ref, hri_ref,
             alp_ref, lam_ref, o_ref):
  y = y_ref[...]
  x = x_ref[...]
  s1 = agg_ref[0]
  s2 = agg_ref[1]
  d_ri = deg_ref[0]
  d_r = deg_ref[1]
  hr = hr_ref[...]
  hri = hri_ref[...]
  alp = alp_ref[0, 0]
  lam = lam_ref[0, 0]

  dot = functools.partial(
      lax.dot_general, dimension_numbers=(((1,), (0,)), ((), ())),
      preferred_element_type=jnp.float32)
  dott = functools.partial(
      lax.dot_general, dimension_numbers=(((1,), (1,)), ((), ())),
      preferred_element_type=jnp.float32)

  yhr = dott(dot(y, hr), hr)
  yhri = dott(dot(y, hri), hri)
  a1 = dot(s1, hr) + dott(s1, hri)
  a2 = dot(s2, hri) + dott(s2, hr)
  deg = d_r + d_ri

  r = x + a1 + a2 - d_r * yhr - d_ri * yhri
  r = (1.0 - alp) * y + (alp * lam) * r / (1.0 + lam * deg)
  o_ref[...] = jnp.maximum(r, 0.0)


@jax.jit
def _tc_combine(y, x, agg, deg, hr, hri, alp, lam):
  blk = 1000
  grid = N // blk
  return pl.pallas_call(
      _tc_body,
      grid=(grid,),
      in_specs=[
          pl.BlockSpec((blk, D), lambda i: (i, 0)),
          pl.BlockSpec((blk, D), lambda i: (i, 0)),
          pl.BlockSpec((NC, blk, D), lambda i: (0, i, 0)),
          pl.BlockSpec((NC, blk, 1), lambda i: (0, i, 0)),
          pl.BlockSpec((D, D), lambda i: (0, 0)),
          pl.BlockSpec((D, D), lambda i: (0, 0)),
          pl.BlockSpec(memory_space=pltpu.SMEM),
          pl.BlockSpec(memory_space=pltpu.SMEM),
      ],
      out_specs=pl.BlockSpec((blk, D), lambda i: (i, 0)),
      out_shape=jax.ShapeDtypeStruct((N, D), jnp.float32),
  )(y, x, agg, deg, hr, hri, alp, lam)


def kernel(Y, X, edge_index, w_r, w_rinv, H_r, H_rinv, alp, lam):
  src = edge_index[0]
  dst = edge_index[1]
  npad = E_PAD - E
  cyc = jnp.arange(npad, dtype=jnp.int32) % (N_T - N)
  pad_g = cyc
  pad_s = N + cyc
  pad_w = jnp.zeros((npad,), dtype=jnp.float32)

  src_g = jnp.concatenate([src, pad_g])
  dst_g = jnp.concatenate([dst, pad_g])
  src_s = jnp.concatenate([src, pad_s])
  dst_s = jnp.concatenate([dst, pad_s])
  wr_b = lax.bitcast_convert_type(
      jnp.concatenate([w_r[:, 0], pad_w]), jnp.int32)
  wri_b = lax.bitcast_convert_type(
      jnp.concatenate([w_rinv[:, 0], pad_w]), jnp.int32)

  shape4 = (NC, NS, NCH, C)
  gidx = jnp.stack([src_g, dst_g]).reshape(shape4)
  sidx = jnp.stack([dst_s, src_s]).reshape(shape4)
  wb = jnp.stack([wr_b, wri_b]).reshape(shape4)
  pk = jnp.stack([gidx, sidx, wb], axis=3)

  agg, deg = _sc_segment_sums(Y, pk)
  deg3 = deg.reshape(NC, HR * D, 1)
  alp11 = jnp.reshape(alp, (1, 1)).astype(jnp.float32)
  lam11 = jnp.reshape(lam, (1, 1)).astype(jnp.float32)
  return _tc_combine(Y, X, agg, deg3, H_r, H_rinv, alp11, lam11)
